# Initial kernel scaffold; baseline (speedup 1.0000x reference)
import jax
import jax.numpy as jnp
from jax.experimental import pallas as pl


def kernel(graph, r_ij, res_emb, W1, b1, W2, b2, W3, Wv):
    raise NotImplementedError("write your pallas kernel here")


# speedup vs baseline: 14.6384x; 14.6384x over previous
import functools

import jax
import jax.numpy as jnp
import numpy as np
from jax import lax
from jax.experimental import pallas as pl
from jax.experimental.pallas import tpu as pltpu
from jax.experimental.pallas import tpu_sc as plsc

R0 = 5.0
DIM_A = 128
DIM_V = 64
DV3 = 3 * DIM_V
N_NODES = 10000
N_EDGES = 320000
LEAK = 0.1

NC = 2
NS = 16
NW = NC * NS

CH = 128

EPW_G = N_EDGES // NW
NCH_G = EPW_G // CH
TAIL_G = EPW_G - NCH_G * CH

CH_S = 80
EPT_S = N_EDGES // NS
NCH_S = EPT_S // CH_S

NODE0 = 5008
ACC_ROWS = NODE0 + 8
DUMP = NODE0
NPT = 312
ZREM = ACC_ROWS - NS * NPT


def _gather_body(dst_hbm, emb_hbm, out_hbm, idx_v, idx_t, rows_v, rows_t):
    c = lax.axis_index("c")
    s = lax.axis_index("s")
    wid = s * NC + c
    base = wid * EPW_G

    def chunk(i, carry):
        off = base + i * CH
        pltpu.sync_copy(dst_hbm.at[pl.ds(off, CH)], idx_v)
        pltpu.sync_copy(emb_hbm.at[idx_v], rows_v)
        pltpu.sync_copy(rows_v, out_hbm.at[pl.ds(off, CH)])
        return carry

    lax.fori_loop(0, NCH_G, chunk, 0)
    off = base + NCH_G * CH
    pltpu.sync_copy(dst_hbm.at[pl.ds(off, TAIL_G)], idx_t)
    pltpu.sync_copy(emb_hbm.at[idx_t], rows_t)
    pltpu.sync_copy(rows_t, out_hbm.at[pl.ds(off, TAIL_G)])


BE = 2000


def _tc_body(rij_ref, emb_ref, w1, b1r, w2, b2r, w3, wvt, phi_ref, gv_ref):
    r = rij_ref[...]
    emb = emb_ref[...]
    d = jnp.sqrt(jnp.sum(r * r, axis=1, keepdims=True))
    k = lax.broadcasted_iota(jnp.int32, (1, DIM_A), 1)
    j = k % (DIM_A // 2)
    coef = (1 + j // 2).astype(jnp.float32) * (np.pi / R0)
    shift = jnp.where(k < DIM_A // 2, 0.0, np.pi / 2).astype(jnp.float32)
    enc = jnp.cos(d * coef - shift) + emb

    h = jnp.dot(enc, w1[...], preferred_element_type=jnp.float32) + b1r[...]
    h = jnp.where(h >= 0, h, LEAK * h)
    h = jnp.dot(h, w2[...], preferred_element_type=jnp.float32) + b2r[...]
    h = jnp.where(h >= 0, h, LEAK * h)
    rad = jnp.dot(h, w3[...], preferred_element_type=jnp.float32)
    g = jnp.dot(rad, wvt[...], preferred_element_type=jnp.float32)

    x = r * (7.0 / R0)
    n2 = jnp.sum(x * x, axis=1, keepdims=True)
    rs = x / jnp.sqrt(1.0 + n2)

    phi_ref[...] = rad
    gv_ref[...] = jnp.concatenate(
        [g * rs[:, 0:1], g * rs[:, 1:2], g * rs[:, 2:3]], axis=1)


_tc_call = pl.pallas_call(
    _tc_body,
    grid=(N_EDGES // BE,),
    in_specs=[
        pl.BlockSpec((BE, 3), lambda i: (i, 0)),
        pl.BlockSpec((BE, DIM_A), lambda i: (i, 0)),
        pl.BlockSpec((DIM_A, DIM_A), lambda i: (0, 0)),
        pl.BlockSpec((1, DIM_A), lambda i: (0, 0)),
        pl.BlockSpec((DIM_A, DIM_A), lambda i: (0, 0)),
        pl.BlockSpec((1, DIM_A), lambda i: (0, 0)),
        pl.BlockSpec((DIM_A, DIM_A), lambda i: (0, 0)),
        pl.BlockSpec((DIM_A, DIM_V), lambda i: (0, 0)),
    ],
    out_specs=[
        pl.BlockSpec((BE, DIM_A), lambda i: (i, 0)),
        pl.BlockSpec((BE, DV3), lambda i: (i, 0)),
    ],
    out_shape=[
        jax.ShapeDtypeStruct((N_EDGES, DIM_A), jnp.float32),
        jax.ShapeDtypeStruct((N_EDGES, DV3), jnp.float32),
    ],
)


def _zero_buf(buf, rows, lanes):
    def zrow(i, carry):
        for j in range(lanes // 16):
            buf[i, pl.ds(j * 16, 16)] = jnp.zeros((16,), jnp.float32)
        return carry

    lax.fori_loop(0, rows, zrow, 0)


def _remap_idx(sidx, sidx2, base, limit):
    for v in range(CH_S // 16):
        x = sidx[pl.ds(v * 16, 16)]
        y = x - base
        valid = (y >= 0) & (y < limit)
        sidx2[pl.ds(v * 16, 16)] = jnp.where(valid, y, DUMP)


def _scatter_body(src_hbm, phi_hbm, gv_hbm, outa, outv,
                  sidx, sidx2, bufa, bufv, acc_a, acc_v):
    c = lax.axis_index("c")
    s = lax.axis_index("s")
    base = c * NODE0
    limit = NODE0 - 16 * c

    _zero_buf(bufa, CH_S, DIM_A)
    _zero_buf(bufv, CH_S, DV3)
    r0 = s * NPT
    for o in (0, 80, 160, 240):
        n = min(CH_S, NPT - o)
        pltpu.sync_copy(bufa.at[pl.ds(0, n)], acc_a.at[pl.ds(r0 + o, n)])
        pltpu.sync_copy(bufv.at[pl.ds(0, n)], acc_v.at[pl.ds(r0 + o, n)])

    @pl.when(s == 0)
    def _():
        pltpu.sync_copy(bufa.at[pl.ds(0, ZREM)],
                        acc_a.at[pl.ds(NS * NPT, ZREM)])
        pltpu.sync_copy(bufv.at[pl.ds(0, ZREM)],
                        acc_v.at[pl.ds(NS * NPT, ZREM)])

    plsc.subcore_barrier()

    ebase = s * EPT_S

    def chunk(i, carry):
        off = ebase + i * CH_S
        pltpu.sync_copy(src_hbm.at[s, pl.ds(i * CH_S, CH_S)], sidx)
        _remap_idx(sidx, sidx2, base, limit)
        pltpu.sync_copy(phi_hbm.at[pl.ds(off, CH_S)], bufa)
        pltpu.sync_copy(bufa, acc_a.at[sidx2], add=True)
        pltpu.sync_copy(gv_hbm.at[pl.ds(off, CH_S)], bufv)
        pltpu.sync_copy(bufv, acc_v.at[sidx2], add=True)
        return carry

    lax.fori_loop(0, NCH_S, chunk, 0)

    plsc.subcore_barrier()

    pltpu.sync_copy(acc_a.at[pl.ds(r0, NPT)], outa.at[pl.ds(base + r0, NPT)])
    pltpu.sync_copy(acc_v.at[pl.ds(r0, NPT)], outv.at[pl.ds(base + r0, NPT)])

    @pl.when((s == 0) & (c == 0))
    def _():
        pltpu.sync_copy(acc_a.at[pl.ds(NS * NPT, 16)],
                        outa.at[pl.ds(NS * NPT, 16)])
        pltpu.sync_copy(acc_v.at[pl.ds(NS * NPT, 16)],
                        outv.at[pl.ds(NS * NPT, 16)])


@functools.lru_cache(maxsize=None)
def _sc_kernels():
    mesh = plsc.VectorSubcoreMesh(core_axis_name="c", subcore_axis_name="s",
                                  num_cores=NC, num_subcores=NS)
    gather_k = pl.kernel(
        _gather_body,
        out_type=jax.ShapeDtypeStruct((N_EDGES, DIM_A), jnp.float32),
        mesh=mesh,
        scratch_types=[
            pltpu.VMEM((CH,), jnp.int32),
            pltpu.VMEM((TAIL_G,), jnp.int32),
            pltpu.VMEM((CH, DIM_A), jnp.float32),
            pltpu.VMEM((TAIL_G, DIM_A), jnp.float32),
        ],
    )
    scatter_k = pl.kernel(
        _scatter_body,
        compiler_params=pltpu.CompilerParams(use_tc_tiling_on_sc=False),
        out_type=[
            jax.ShapeDtypeStruct((N_NODES, DIM_A), jnp.float32),
            jax.ShapeDtypeStruct((N_NODES, DV3), jnp.float32),
        ],
        mesh=mesh,
        scratch_types=[
            pltpu.VMEM((CH_S,), jnp.int32),
            pltpu.VMEM((CH_S,), jnp.int32),
            pltpu.VMEM((CH_S, DIM_A), jnp.float32),
            pltpu.VMEM((CH_S, DV3), jnp.float32),
            pltpu.VMEM_SHARED((ACC_ROWS, DIM_A), jnp.float32),
            pltpu.VMEM_SHARED((ACC_ROWS, DV3), jnp.float32),
        ],
    )
    return gather_k, scatter_k


def kernel(graph, r_ij, res_emb, W1, b1, W2, b2, W3, Wv):
    _gather_k, _scatter_k = _sc_kernels()
    src = graph[0]
    dst = graph[1]
    emb_j = _gather_k(dst, res_emb)
    phi, gv = _tc_call(r_ij, emb_j, W1.T, b1[None, :], W2.T, b2[None, :],
                       W3.T, Wv.T)
    a_acc, v_acc = _scatter_k(src.reshape(NS, EPT_S), phi, gv)
    out_v = v_acc.reshape(N_NODES, 3, DIM_V).transpose(0, 2, 1)
    return (a_acc, out_v)


# speedup vs baseline: 25.4040x; 1.7354x over previous
import functools

import jax
import jax.numpy as jnp
import numpy as np
from jax import lax
from jax.experimental import pallas as pl
from jax.experimental.pallas import tpu as pltpu
from jax.experimental.pallas import tpu_sc as plsc

R0 = 5.0
DIM_A = 128
DIM_V = 64
DV3 = 3 * DIM_V
N_NODES = 10000
N_EDGES = 320000
LEAK = 0.1

NC = 2
NS = 16
NW = NC * NS

CH = 128

EPW_G = N_EDGES // NW
NCH_G = EPW_G // CH
TAIL_G = EPW_G - NCH_G * CH

CH_S = 32
EPT_S = N_EDGES // NS
NCH_S = EPT_S // CH_S

NODE0 = 5008
ACC_ROWS = NODE0 + 8
DUMP = NODE0
NPT = 312
ZREM = ACC_ROWS - NS * NPT

_CC = (0.9999994437335172, -19.739034402900092, 64.93061469583039,
       -85.29598973511935, 58.912659471953766, -21.283218653850707)
_RND = 12582912.0


def _gather_body(dst_hbm, emb_hbm, out_hbm, idx_v, idx_t, rows_v, rows_t):
    c = lax.axis_index("c")
    s = lax.axis_index("s")
    wid = s * NC + c
    base = wid * EPW_G

    def chunk(i, carry):
        off = base + i * CH
        pltpu.sync_copy(dst_hbm.at[pl.ds(off, CH)], idx_v)
        pltpu.sync_copy(emb_hbm.at[idx_v], rows_v)
        pltpu.sync_copy(rows_v, out_hbm.at[pl.ds(off, CH)])
        return carry

    lax.fori_loop(0, NCH_G, chunk, 0)
    off = base + NCH_G * CH
    pltpu.sync_copy(dst_hbm.at[pl.ds(off, TAIL_G)], idx_t)
    pltpu.sync_copy(emb_hbm.at[idx_t], rows_t)
    pltpu.sync_copy(rows_t, out_hbm.at[pl.ds(off, TAIL_G)])


BE = 2000


def _tc_body(rij_ref, emb_ref, w1, b1r, w2, b2r, w3, wvt,
             phi_ref, gxy_ref, gz_ref):
    r = rij_ref[...]
    emb = emb_ref[...]
    rx = r[:, 0:1]
    ry = r[:, 1:2]
    rz = r[:, 2:3]
    d = jnp.sqrt(rx * rx + ry * ry + rz * rz)

    k = lax.broadcasted_iota(jnp.int32, (1, DIM_A), 1)
    j = k % (DIM_A // 2)
    m = (1 + j // 2).astype(jnp.float32)
    ct = m * (1.0 / (2.0 * R0))
    st = jnp.where(k < DIM_A // 2, 0.0, 0.25).astype(jnp.float32)
    y = d * ct - st
    f = y - jnp.floor(y + 0.5)
    u = f * f
    p = jnp.float32(_CC[5])
    for cc in (_CC[4], _CC[3], _CC[2], _CC[1], _CC[0]):
        p = p * u + cc
    enc = p + emb

    h = jnp.dot(enc, w1[...], preferred_element_type=jnp.float32) + b1r[...]
    h = jnp.where(h >= 0, h, LEAK * h)
    h = jnp.dot(h, w2[...], preferred_element_type=jnp.float32) + b2r[...]
    h = jnp.where(h >= 0, h, LEAK * h)
    rad = jnp.dot(h, w3[...], preferred_element_type=jnp.float32)
    g = jnp.dot(rad, wvt[...], preferred_element_type=jnp.float32)

    sc = 7.0 / R0
    n2 = (rx * rx + ry * ry + rz * rz) * (sc * sc)
    inv = sc / jnp.sqrt(1.0 + n2)
    phi_ref[...] = rad
    gxy_ref[...] = jnp.concatenate([g * (rx * inv), g * (ry * inv)], axis=1)
    gz_ref[...] = jnp.concatenate(
        [g * (rz * inv), jnp.zeros((BE, DIM_V), jnp.float32)], axis=1)


_tc_call = pl.pallas_call(
    _tc_body,
    grid=(N_EDGES // BE,),
    in_specs=[
        pl.BlockSpec((BE, 3), lambda i: (i, 0)),
        pl.BlockSpec((BE, DIM_A), lambda i: (i, 0)),
        pl.BlockSpec((DIM_A, DIM_A), lambda i: (0, 0)),
        pl.BlockSpec((1, DIM_A), lambda i: (0, 0)),
        pl.BlockSpec((DIM_A, DIM_A), lambda i: (0, 0)),
        pl.BlockSpec((1, DIM_A), lambda i: (0, 0)),
        pl.BlockSpec((DIM_A, DIM_A), lambda i: (0, 0)),
        pl.BlockSpec((DIM_A, DIM_V), lambda i: (0, 0)),
    ],
    out_specs=[
        pl.BlockSpec((BE, DIM_A), lambda i: (i, 0)),
        pl.BlockSpec((BE, DIM_A), lambda i: (i, 0)),
        pl.BlockSpec((BE, DIM_A), lambda i: (i, 0)),
    ],
    out_shape=[
        jax.ShapeDtypeStruct((N_EDGES, DIM_A), jnp.float32),
        jax.ShapeDtypeStruct((N_EDGES, DIM_A), jnp.float32),
        jax.ShapeDtypeStruct((N_EDGES, DIM_A), jnp.float32),
    ],
)


def _zero_buf(buf, rows, lanes):
    def zrow(i, carry):
        for j in range(lanes // 16):
            buf[i, pl.ds(j * 16, 16)] = jnp.zeros((16,), jnp.float32)
        return carry

    lax.fori_loop(0, rows, zrow, 0)


def _scatter_body(src_hbm, phi_hbm, gxy_hbm, gz_hbm, outa, outv,
                  sidx0, sidx1, sx0, sx1, bufp0, bufp1, bufv0, bufv1,
                  acc_a, acc_v,
                  li0, li1, lp0, lp1, lv0, lv1, lz0, lz1):
    c = lax.axis_index("c")
    s = lax.axis_index("s")
    base = c * NODE0
    limit = NODE0 - 16 * c

    sidx = (sidx0, sidx1)
    sx = (sx0, sx1)
    bufp = (bufp0, bufp1)
    bufv = (bufv0, bufv1)
    li = (li0, li1)
    lp = (lp0, lp1)
    lv = (lv0, lv1)
    lz = (lz0, lz1)

    _zero_buf(bufp0, CH_S, DIM_A)
    _zero_buf(bufv0, CH_S, DV3)
    r0 = s * NPT
    for o in range(0, NPT, CH_S):
        n = min(CH_S, NPT - o)
        pltpu.sync_copy(bufp0.at[pl.ds(0, n)], acc_a.at[pl.ds(r0 + o, n)])
        pltpu.sync_copy(bufv0.at[pl.ds(0, n)], acc_v.at[pl.ds(r0 + o, n)])

    @pl.when(s == 0)
    def _():
        pltpu.sync_copy(bufp0.at[pl.ds(0, ZREM)],
                        acc_a.at[pl.ds(NS * NPT, ZREM)])
        pltpu.sync_copy(bufv0.at[pl.ds(0, ZREM)],
                        acc_v.at[pl.ds(NS * NPT, ZREM)])

    plsc.subcore_barrier()

    ebase = s * EPT_S

    def fire_loads(ci, b):
        off = ebase + ci * CH_S
        pltpu.async_copy(src_hbm.at[s, pl.ds(ci * CH_S, CH_S)], sidx[b], li[b])
        pltpu.async_copy(phi_hbm.at[pl.ds(off, CH_S)], bufp[b], lp[b])
        pltpu.async_copy(gxy_hbm.at[pl.ds(off, CH_S)],
                         bufv[b].at[:, pl.ds(0, DIM_A)], lv[b])
        pltpu.async_copy(gz_hbm.at[pl.ds(off, CH_S), pl.ds(0, DIM_V)],
                         bufv[b].at[:, pl.ds(DIM_A, DIM_V)], lz[b])

    def process(ci, b):
        off = ebase + ci * CH_S
        pltpu.make_async_copy(src_hbm.at[s, pl.ds(ci * CH_S, CH_S)],
                              sidx[b], li[b]).wait()
        for v in range(CH_S // 16):
            x = sidx[b][pl.ds(v * 16, 16)]
            yy = x - base
            valid = (yy >= 0) & (yy < limit)
            sx[b][pl.ds(v * 16, 16)] = jnp.where(valid, yy, DUMP)
        pltpu.make_async_copy(phi_hbm.at[pl.ds(off, CH_S)],
                              bufp[b], lp[b]).wait()
        pltpu.sync_copy(bufp[b], acc_a.at[sx[b]], add=True)
        pltpu.make_async_copy(gxy_hbm.at[pl.ds(off, CH_S)],
                              bufv[b].at[:, pl.ds(0, DIM_A)], lv[b]).wait()
        pltpu.make_async_copy(gz_hbm.at[pl.ds(off, CH_S), pl.ds(0, DIM_V)],
                              bufv[b].at[:, pl.ds(DIM_A, DIM_V)], lz[b]).wait()
        pltpu.sync_copy(bufv[b], acc_v.at[sx[b]], add=True)

        @pl.when(ci + 2 < NCH_S)
        def _():
            fire_loads(ci + 2, b)

    fire_loads(0, 0)
    fire_loads(1, 1)

    def pair(i2, carry):
        process(2 * i2, 0)
        process(2 * i2 + 1, 1)
        return carry

    lax.fori_loop(0, NCH_S // 2, pair, 0)
    process(NCH_S - 1, 0)

    plsc.subcore_barrier()

    pltpu.sync_copy(acc_a.at[pl.ds(r0, NPT)], outa.at[pl.ds(base + r0, NPT)])
    pltpu.sync_copy(acc_v.at[pl.ds(r0, NPT)], outv.at[pl.ds(base + r0, NPT)])

    @pl.when((s == 0) & (c == 0))
    def _():
        pltpu.sync_copy(acc_a.at[pl.ds(NS * NPT, 16)],
                        outa.at[pl.ds(NS * NPT, 16)])
        pltpu.sync_copy(acc_v.at[pl.ds(NS * NPT, 16)],
                        outv.at[pl.ds(NS * NPT, 16)])


@functools.lru_cache(maxsize=None)
def _sc_kernels():
    mesh = plsc.VectorSubcoreMesh(core_axis_name="c", subcore_axis_name="s",
                                  num_cores=NC, num_subcores=NS)
    gather_k = pl.kernel(
        _gather_body,
        out_type=jax.ShapeDtypeStruct((N_EDGES, DIM_A), jnp.float32),
        mesh=mesh,
        scratch_types=[
            pltpu.VMEM((CH,), jnp.int32),
            pltpu.VMEM((TAIL_G,), jnp.int32),
            pltpu.VMEM((CH, DIM_A), jnp.float32),
            pltpu.VMEM((TAIL_G, DIM_A), jnp.float32),
        ],
    )
    scatter_k = pl.kernel(
        _scatter_body,
        compiler_params=pltpu.CompilerParams(use_tc_tiling_on_sc=False),
        out_type=[
            jax.ShapeDtypeStruct((N_NODES, DIM_A), jnp.float32),
            jax.ShapeDtypeStruct((N_NODES, DV3), jnp.float32),
        ],
        mesh=mesh,
        scratch_types=[
            pltpu.VMEM((CH_S,), jnp.int32),
            pltpu.VMEM((CH_S,), jnp.int32),
            pltpu.VMEM((CH_S,), jnp.int32),
            pltpu.VMEM((CH_S,), jnp.int32),
            pltpu.VMEM((CH_S, DIM_A), jnp.float32),
            pltpu.VMEM((CH_S, DIM_A), jnp.float32),
            pltpu.VMEM((CH_S, DV3), jnp.float32),
            pltpu.VMEM((CH_S, DV3), jnp.float32),
            pltpu.VMEM_SHARED((ACC_ROWS, DIM_A), jnp.float32),
            pltpu.VMEM_SHARED((ACC_ROWS, DV3), jnp.float32),
        ] + [pltpu.SemaphoreType.DMA] * 8,
    )
    return gather_k, scatter_k


def kernel(graph, r_ij, res_emb, W1, b1, W2, b2, W3, Wv):
    _gather_k, _scatter_k = _sc_kernels()
    src = graph[0]
    dst = graph[1]
    emb_j = _gather_k(dst, res_emb)
    phi, gxy, gz = _tc_call(r_ij, emb_j, W1.T, b1[None, :], W2.T, b2[None, :],
                            W3.T, Wv.T)
    a_acc, v_acc = _scatter_k(src.reshape(NS, EPT_S), phi, gxy, gz)
    out_v = v_acc.reshape(N_NODES, 3, DIM_V).transpose(0, 2, 1)
    return (a_acc, out_v)


# speedup vs baseline: 26.7356x; 1.0524x over previous
import functools

import jax
import jax.numpy as jnp
import numpy as np
from jax import lax
from jax.experimental import pallas as pl
from jax.experimental.pallas import tpu as pltpu
from jax.experimental.pallas import tpu_sc as plsc

R0 = 5.0
DIM_A = 128
DIM_V = 64
DV3 = 3 * DIM_V
N_NODES = 10000
N_EDGES = 320000
LEAK = 0.1

NC = 2
NS = 16
NW = NC * NS

CH = 128

EPW_G = N_EDGES // NW
NCH_G = EPW_G // CH
TAIL_G = EPW_G - NCH_G * CH

CH_S = 32
EPT_S = N_EDGES // NS
NCH_S = EPT_S // CH_S

NODE0 = 5008
ACC_ROWS = NODE0 + 8
DUMP = NODE0
NPT = 312
ZREM = ACC_ROWS - NS * NPT

_CC = (0.9999994437335172, -19.739034402900092, 64.93061469583039,
       -85.29598973511935, 58.912659471953766, -21.283218653850707)
_RND = 12582912.0


def _gather_body(dst_hbm, emb_hbm, out_hbm, idx0, idx1, idx_t,
                 rows0, rows1, rows_t, ji0, ji1, js0, js1):
    c = lax.axis_index("c")
    s = lax.axis_index("s")
    wid = s * NC + c
    base = wid * EPW_G
    idx = (idx0, idx1)
    rows = (rows0, rows1)
    ji = (ji0, ji1)
    js = (js0, js1)

    def prologue(ci, b):
        off = base + ci * CH
        pltpu.sync_copy(dst_hbm.at[pl.ds(off, CH)], idx[b])
        pltpu.sync_copy(emb_hbm.at[idx[b]], rows[b])
        pltpu.async_copy(rows[b], out_hbm.at[pl.ds(off, CH)], js[b])
        pltpu.async_copy(dst_hbm.at[pl.ds(off + 2 * CH, CH)], idx[b], ji[b])

    prologue(0, 0)
    prologue(1, 1)

    def step(ci, b):
        off = base + ci * CH
        pltpu.make_async_copy(dst_hbm.at[pl.ds(off, CH)],
                              idx[b], ji[b]).wait()
        pltpu.make_async_copy(rows[b],
                              out_hbm.at[pl.ds(off, CH)], js[b]).wait()
        pltpu.sync_copy(emb_hbm.at[idx[b]], rows[b])
        pltpu.async_copy(rows[b], out_hbm.at[pl.ds(off, CH)], js[b])

        @pl.when(ci + 2 < NCH_G)
        def _():
            pltpu.async_copy(dst_hbm.at[pl.ds(off + 2 * CH, CH)],
                             idx[b], ji[b])

    def chunk(i, carry):
        ci = 2 * i + 2
        step(ci, 0)
        step(ci + 1, 1)
        return carry

    lax.fori_loop(0, (NCH_G - 2) // 2, chunk, 0)
    for b in (0, 1):
        off = base + (NCH_G - 2 + b) * CH
        pltpu.make_async_copy(rows[b],
                              out_hbm.at[pl.ds(off, CH)], js[b]).wait()
    off = base + NCH_G * CH
    pltpu.sync_copy(dst_hbm.at[pl.ds(off, TAIL_G)], idx_t)
    pltpu.sync_copy(emb_hbm.at[idx_t], rows_t)
    pltpu.sync_copy(rows_t, out_hbm.at[pl.ds(off, TAIL_G)])


BE = 4000


def _tc_body(rij_ref, emb_ref, w1, b1r, w2, b2r, w3, wvt,
             phi_ref, gxy_ref, gz_ref):
    r = rij_ref[...]
    emb = emb_ref[...]
    rx = r[:, 0:1]
    ry = r[:, 1:2]
    rz = r[:, 2:3]
    sc = 7.0 / R0
    d2 = rx * rx + ry * ry + rz * rz
    irt = lax.rsqrt(jnp.concatenate(
        [jnp.maximum(d2, 1e-30), 1.0 + (sc * sc) * d2], axis=1))
    d = d2 * irt[:, 0:1]
    inv = sc * irt[:, 1:2]

    k = lax.broadcasted_iota(jnp.int32, (1, DIM_A), 1)
    j = k % (DIM_A // 2)
    m = (1 + j // 2).astype(jnp.float32)
    ct = m * (1.0 / (2.0 * R0))
    st = jnp.where(k < DIM_A // 2, 0.0, 0.25).astype(jnp.float32)
    y = d * ct - st
    f = y - jnp.floor(y + 0.5)
    u = f * f
    p = jnp.float32(_CC[5])
    for cc in (_CC[4], _CC[3], _CC[2], _CC[1], _CC[0]):
        p = p * u + cc
    enc = p + emb

    h = jnp.dot(enc, w1[...], preferred_element_type=jnp.float32) + b1r[...]
    h = jnp.where(h >= 0, h, LEAK * h)
    h = jnp.dot(h, w2[...], preferred_element_type=jnp.float32) + b2r[...]
    h = jnp.where(h >= 0, h, LEAK * h)
    rad = jnp.dot(h, w3[...], preferred_element_type=jnp.float32)
    g = jnp.dot(rad, wvt[...], preferred_element_type=jnp.float32)

    phi_ref[...] = rad
    gxy_ref[...] = jnp.concatenate([g * (rx * inv), g * (ry * inv)], axis=1)
    gz_ref[...] = jnp.concatenate(
        [g * (rz * inv), jnp.zeros((BE, DIM_V), jnp.float32)], axis=1)


_tc_call = pl.pallas_call(
    _tc_body,
    grid=(N_EDGES // BE,),
    in_specs=[
        pl.BlockSpec((BE, 3), lambda i: (i, 0)),
        pl.BlockSpec((BE, DIM_A), lambda i: (i, 0)),
        pl.BlockSpec((DIM_A, DIM_A), lambda i: (0, 0)),
        pl.BlockSpec((1, DIM_A), lambda i: (0, 0)),
        pl.BlockSpec((DIM_A, DIM_A), lambda i: (0, 0)),
        pl.BlockSpec((1, DIM_A), lambda i: (0, 0)),
        pl.BlockSpec((DIM_A, DIM_A), lambda i: (0, 0)),
        pl.BlockSpec((DIM_A, DIM_V), lambda i: (0, 0)),
    ],
    out_specs=[
        pl.BlockSpec((BE, DIM_A), lambda i: (i, 0)),
        pl.BlockSpec((BE, DIM_A), lambda i: (i, 0)),
        pl.BlockSpec((BE, DIM_A), lambda i: (i, 0)),
    ],
    out_shape=[
        jax.ShapeDtypeStruct((N_EDGES, DIM_A), jnp.float32),
        jax.ShapeDtypeStruct((N_EDGES, DIM_A), jnp.float32),
        jax.ShapeDtypeStruct((N_EDGES, DIM_A), jnp.float32),
    ],
)


def _zero_buf(buf, rows, lanes):
    def zrow(i, carry):
        for j in range(lanes // 16):
            buf[i, pl.ds(j * 16, 16)] = jnp.zeros((16,), jnp.float32)
        return carry

    lax.fori_loop(0, rows, zrow, 0)


def _scatter_body(src_hbm, phi_hbm, gxy_hbm, gz_hbm, outa, outv,
                  sidx0, sidx1, sx0, sx1, bufp0, bufp1, bufv0, bufv1,
                  acc_a, acc_v,
                  li0, li1, lp0, lp1, lv0, lv1, lz0, lz1):
    c = lax.axis_index("c")
    s = lax.axis_index("s")
    base = c * NODE0
    limit = NODE0 - 16 * c

    sidx = (sidx0, sidx1)
    sx = (sx0, sx1)
    bufp = (bufp0, bufp1)
    bufv = (bufv0, bufv1)
    li = (li0, li1)
    lp = (lp0, lp1)
    lv = (lv0, lv1)
    lz = (lz0, lz1)

    _zero_buf(bufp0, CH_S, DIM_A)
    _zero_buf(bufv0, CH_S, DV3)
    r0 = s * NPT
    for o in range(0, NPT, CH_S):
        n = min(CH_S, NPT - o)
        pltpu.sync_copy(bufp0.at[pl.ds(0, n)], acc_a.at[pl.ds(r0 + o, n)])
        pltpu.sync_copy(bufv0.at[pl.ds(0, n)], acc_v.at[pl.ds(r0 + o, n)])

    @pl.when(s == 0)
    def _():
        pltpu.sync_copy(bufp0.at[pl.ds(0, ZREM)],
                        acc_a.at[pl.ds(NS * NPT, ZREM)])
        pltpu.sync_copy(bufv0.at[pl.ds(0, ZREM)],
                        acc_v.at[pl.ds(NS * NPT, ZREM)])

    plsc.subcore_barrier()

    ebase = s * EPT_S

    def fire_loads(ci, b):
        off = ebase + ci * CH_S
        pltpu.async_copy(src_hbm.at[s, pl.ds(ci * CH_S, CH_S)], sidx[b], li[b])
        pltpu.async_copy(phi_hbm.at[pl.ds(off, CH_S)], bufp[b], lp[b])
        pltpu.async_copy(gxy_hbm.at[pl.ds(off, CH_S)],
                         bufv[b].at[:, pl.ds(0, DIM_A)], lv[b])
        pltpu.async_copy(gz_hbm.at[pl.ds(off, CH_S), pl.ds(0, DIM_V)],
                         bufv[b].at[:, pl.ds(DIM_A, DIM_V)], lz[b])

    def process(ci, b):
        off = ebase + ci * CH_S
        pltpu.make_async_copy(src_hbm.at[s, pl.ds(ci * CH_S, CH_S)],
                              sidx[b], li[b]).wait()
        for v in range(CH_S // 16):
            x = sidx[b][pl.ds(v * 16, 16)]
            yy = x - base
            valid = (yy >= 0) & (yy < limit)
            sx[b][pl.ds(v * 16, 16)] = jnp.where(valid, yy, DUMP)
        pltpu.make_async_copy(phi_hbm.at[pl.ds(off, CH_S)],
                              bufp[b], lp[b]).wait()
        pltpu.sync_copy(bufp[b], acc_a.at[sx[b]], add=True)
        pltpu.make_async_copy(gxy_hbm.at[pl.ds(off, CH_S)],
                              bufv[b].at[:, pl.ds(0, DIM_A)], lv[b]).wait()
        pltpu.make_async_copy(gz_hbm.at[pl.ds(off, CH_S), pl.ds(0, DIM_V)],
                              bufv[b].at[:, pl.ds(DIM_A, DIM_V)], lz[b]).wait()
        pltpu.sync_copy(bufv[b], acc_v.at[sx[b]], add=True)

        @pl.when(ci + 2 < NCH_S)
        def _():
            fire_loads(ci + 2, b)

    fire_loads(0, 0)
    fire_loads(1, 1)

    def pair(i2, carry):
        process(2 * i2, 0)
        process(2 * i2 + 1, 1)
        return carry

    lax.fori_loop(0, NCH_S // 2, pair, 0)
    process(NCH_S - 1, 0)

    plsc.subcore_barrier()

    pltpu.sync_copy(acc_a.at[pl.ds(r0, NPT)], outa.at[pl.ds(base + r0, NPT)])
    pltpu.sync_copy(acc_v.at[pl.ds(r0, NPT)], outv.at[pl.ds(base + r0, NPT)])

    @pl.when((s == 0) & (c == 0))
    def _():
        pltpu.sync_copy(acc_a.at[pl.ds(NS * NPT, 16)],
                        outa.at[pl.ds(NS * NPT, 16)])
        pltpu.sync_copy(acc_v.at[pl.ds(NS * NPT, 16)],
                        outv.at[pl.ds(NS * NPT, 16)])


@functools.lru_cache(maxsize=None)
def _sc_kernels():
    mesh = plsc.VectorSubcoreMesh(core_axis_name="c", subcore_axis_name="s",
                                  num_cores=NC, num_subcores=NS)
    gather_k = pl.kernel(
        _gather_body,
        out_type=jax.ShapeDtypeStruct((N_EDGES, DIM_A), jnp.float32),
        mesh=mesh,
        scratch_types=[
            pltpu.VMEM((CH,), jnp.int32),
            pltpu.VMEM((CH,), jnp.int32),
            pltpu.VMEM((TAIL_G,), jnp.int32),
            pltpu.VMEM((CH, DIM_A), jnp.float32),
            pltpu.VMEM((CH, DIM_A), jnp.float32),
            pltpu.VMEM((TAIL_G, DIM_A), jnp.float32),
        ] + [pltpu.SemaphoreType.DMA] * 4,
    )
    scatter_k = pl.kernel(
        _scatter_body,
        compiler_params=pltpu.CompilerParams(use_tc_tiling_on_sc=False),
        out_type=[
            jax.ShapeDtypeStruct((N_NODES, DIM_A), jnp.float32),
            jax.ShapeDtypeStruct((N_NODES, DV3), jnp.float32),
        ],
        mesh=mesh,
        scratch_types=[
            pltpu.VMEM((CH_S,), jnp.int32),
            pltpu.VMEM((CH_S,), jnp.int32),
            pltpu.VMEM((CH_S,), jnp.int32),
            pltpu.VMEM((CH_S,), jnp.int32),
            pltpu.VMEM((CH_S, DIM_A), jnp.float32),
            pltpu.VMEM((CH_S, DIM_A), jnp.float32),
            pltpu.VMEM((CH_S, DV3), jnp.float32),
            pltpu.VMEM((CH_S, DV3), jnp.float32),
            pltpu.VMEM_SHARED((ACC_ROWS, DIM_A), jnp.float32),
            pltpu.VMEM_SHARED((ACC_ROWS, DV3), jnp.float32),
        ] + [pltpu.SemaphoreType.DMA] * 8,
    )
    return gather_k, scatter_k


def kernel(graph, r_ij, res_emb, W1, b1, W2, b2, W3, Wv):
    _gather_k, _scatter_k = _sc_kernels()
    src = graph[0]
    dst = graph[1]
    emb_j = _gather_k(dst, res_emb)
    phi, gxy, gz = _tc_call(r_ij, emb_j, W1.T, b1[None, :], W2.T, b2[None, :],
                            W3.T, Wv.T)
    a_acc, v_acc = _scatter_k(src.reshape(NS, EPT_S), phi, gxy, gz)
    out_v = v_acc.reshape(N_NODES, 3, DIM_V).transpose(0, 2, 1)
    return (a_acc, out_v)
